# baseline (speedup 1.0000x reference)
import functools

import jax
import jax.numpy as jnp
from jax import lax
from jax.experimental import pallas as pl
from jax.experimental.pallas import tpu as pltpu
from jax.experimental.pallas import tpu_sc as plsc

N_NODES = 100000
D = 128
CH = 128
NCH = N_NODES // CH
TAIL = N_NODES - NCH * CH
NC = 2
NS = 16
NW = NC * NS
STEPS = -(-NCH // NW)


def _sc_gather(table, idx):
    mesh = plsc.VectorSubcoreMesh(core_axis_name="c", subcore_axis_name="s")

    @functools.partial(
        pl.kernel,
        out_type=jax.ShapeDtypeStruct((N_NODES, D), jnp.float32),
        mesh=mesh,
        scratch_types=[
            pltpu.VMEM((CH,), jnp.int32),
            pltpu.VMEM((CH, D), jnp.float32),
            pltpu.SemaphoreType.DMA,
        ],
    )
    def k(table_hbm, idx_hbm, out_hbm, idx_v, buf, sem):
        wid = lax.axis_index("s") * NC + lax.axis_index("c")

        def step(t, carry):
            c = wid + t * NW

            @pl.when(c < NCH)
            def _():
                row0 = pl.multiple_of(c * CH, CH)
                pltpu.sync_copy(idx_hbm.at[pl.ds(row0, CH)], idx_v)
                pltpu.async_copy(table_hbm.at[idx_v], buf, sem).wait()
                pltpu.sync_copy(buf, out_hbm.at[pl.ds(row0, CH)])

            return carry

        lax.fori_loop(0, STEPS, step, 0)

        @pl.when(wid == NW - 1)
        def _():
            row0 = NCH * CH
            pltpu.sync_copy(idx_hbm.at[pl.ds(row0, TAIL)],
                            idx_v.at[pl.ds(0, TAIL)])
            pltpu.async_copy(table_hbm.at[idx_v.at[pl.ds(0, TAIL)]],
                             buf.at[pl.ds(0, TAIL)], sem).wait()
            pltpu.sync_copy(buf.at[pl.ds(0, TAIL)],
                            out_hbm.at[pl.ds(row0, TAIL)])

    return k(table, idx)


def kernel(graph_feat, node_segment):
    idx = node_segment.astype(jnp.int32)
    return _sc_gather(graph_feat, idx)


# speedup vs baseline: 1.6602x; 1.6602x over previous
import functools

import jax
import jax.numpy as jnp
from jax import lax
from jax.experimental import pallas as pl
from jax.experimental.pallas import tpu as pltpu
from jax.experimental.pallas import tpu_sc as plsc

N_NODES = 100000
D = 128
CH = 128
NCHUNK = 782
LAST = NCHUNK - 1
LAST_ROW0 = N_NODES - CH
NC = 2
NS = 16
NW = NC * NS
STEPS = 25
STAGE = STEPS * CH
NBUF = 4


def _sc_gather(table, idx):
    mesh = plsc.VectorSubcoreMesh(core_axis_name="c", subcore_axis_name="s")

    @functools.partial(
        pl.kernel,
        out_type=jax.ShapeDtypeStruct((N_NODES, D), jnp.float32),
        mesh=mesh,
        scratch_types=(
            [pltpu.VMEM((STAGE,), jnp.int32)]
            + [pltpu.VMEM((CH, D), jnp.float32) for _ in range(NBUF)]
            + [pltpu.SemaphoreType.DMA for _ in range(2 * NBUF)]
        ),
    )
    def k(table_hbm, idx_hbm, out_hbm, idx_v, *rest):
        bufs = rest[:NBUF]
        gsem = rest[NBUF:2 * NBUF]
        wsem = rest[2 * NBUF:]

        wid = lax.axis_index("s") * NC + lax.axis_index("c")
        start_chunk = 24 * wid + jnp.minimum(wid, 14)
        n_last = jnp.where(wid < 14, STEPS - 1, STEPS - 2)
        base_stage = jnp.where(wid == NW - 1, N_NODES - STAGE,
                               start_chunk * CH)
        base_stage = pl.multiple_of(base_stage, 8)

        def out_row(t):
            c = start_chunk + jnp.minimum(t, n_last)
            return pl.multiple_of(jnp.where(c == LAST, LAST_ROW0, c * CH), 8)

        def gather_desc(t, slot):
            io = pl.multiple_of(out_row(t) - base_stage, 8)
            return pltpu.make_async_copy(
                table_hbm.at[idx_v.at[pl.ds(io, CH)]],
                bufs[slot], gsem[slot])

        def write_desc(t, slot):
            return pltpu.make_async_copy(
                bufs[slot], out_hbm.at[pl.ds(out_row(t), CH)],
                wsem[slot])

        pltpu.sync_copy(idx_hbm.at[pl.ds(base_stage, STAGE)], idx_v)

        for b in range(NBUF):
            gather_desc(b, b).start()

        def step(t, slot):
            gather_desc(t, slot).wait()
            write_desc(t, slot).start()

        def step_refill(t, slot):
            step(t, slot)
            write_desc(t, slot).wait()
            gather_desc(t + NBUF, slot).start()

        def body(g, carry):
            for b in range(NBUF):
                step_refill(g * NBUF + b, b)
            return carry

        lax.fori_loop(0, (STEPS - NBUF) // NBUF, body, 0)
        step_refill(STEPS - NBUF - 1, (STEPS - NBUF - 1) % NBUF)
        for t in range(STEPS - NBUF, STEPS):
            step(t, t % NBUF)
        for t in range(STEPS - NBUF, STEPS):
            write_desc(t, t % NBUF).wait()

    return k(table, idx)


def kernel(graph_feat, node_segment):
    idx = node_segment.astype(jnp.int32)
    return _sc_gather(graph_feat, idx)


# speedup vs baseline: 1.8158x; 1.0937x over previous
import functools

import jax
import jax.numpy as jnp
from jax import lax
from jax.experimental import pallas as pl
from jax.experimental.pallas import tpu as pltpu
from jax.experimental.pallas import tpu_sc as plsc

N_NODES = 100000
D = 128
CH = 128
NCHUNK = 782
LAST = NCHUNK - 1
LAST_ROW0 = N_NODES - CH
NC = 2
NS = 16
NW = NC * NS
STEPS = 25
STAGE = STEPS * CH
NBUF = 6


def _sc_gather(table, idx):
    mesh = plsc.VectorSubcoreMesh(core_axis_name="c", subcore_axis_name="s")

    @functools.partial(
        pl.kernel,
        out_type=jax.ShapeDtypeStruct((N_NODES, D), jnp.float32),
        mesh=mesh,
        scratch_types=(
            [pltpu.VMEM((STAGE,), jnp.int32)]
            + [pltpu.VMEM((CH, D), jnp.float32) for _ in range(NBUF)]
            + [pltpu.SemaphoreType.DMA for _ in range(2 * NBUF)]
        ),
    )
    def k(table_hbm, idx_hbm, out_hbm, idx_v, *rest):
        bufs = rest[:NBUF]
        gsem = rest[NBUF:2 * NBUF]
        wsem = rest[2 * NBUF:]

        wid = lax.axis_index("s") * NC + lax.axis_index("c")
        start_chunk = 24 * wid + jnp.minimum(wid, 14)
        n_last = jnp.where(wid < 14, STEPS - 1, STEPS - 2)
        base_stage = jnp.where(wid == NW - 1, N_NODES - STAGE,
                               start_chunk * CH)
        base_stage = pl.multiple_of(base_stage, 8)

        def out_row(t):
            c = start_chunk + jnp.minimum(t, n_last)
            return pl.multiple_of(jnp.where(c == LAST, LAST_ROW0, c * CH), 8)

        def gather_desc(t, slot):
            io = pl.multiple_of(out_row(t) - base_stage, 8)
            return pltpu.make_async_copy(
                table_hbm.at[idx_v.at[pl.ds(io, CH)]],
                bufs[slot], gsem[slot])

        def write_desc(t, slot):
            return pltpu.make_async_copy(
                bufs[slot], out_hbm.at[pl.ds(out_row(t), CH)],
                wsem[slot])

        pltpu.sync_copy(idx_hbm.at[pl.ds(base_stage, STAGE)], idx_v)

        for b in range(NBUF):
            gather_desc(b, b).start()

        def step(t, slot):
            gather_desc(t, slot).wait()
            write_desc(t, slot).start()

        def step_refill(t, slot):
            step(t, slot)
            write_desc(t, slot).wait()
            gather_desc(t + NBUF, slot).start()

        def body(g, carry):
            for b in range(NBUF):
                step_refill(g * NBUF + b, b)
            return carry

        full = (STEPS - NBUF) // NBUF
        lax.fori_loop(0, full, body, 0)
        for t in range(full * NBUF, STEPS - NBUF):
            step_refill(t, t % NBUF)
        for t in range(STEPS - NBUF, STEPS):
            step(t, t % NBUF)
        for t in range(STEPS - NBUF, STEPS):
            write_desc(t, t % NBUF).wait()

    return k(table, idx)


def kernel(graph_feat, node_segment):
    idx = node_segment.astype(jnp.int32)
    return _sc_gather(graph_feat, idx)
